# Initial kernel scaffold; baseline (speedup 1.0000x reference)
import jax
import jax.numpy as jnp
from jax.experimental import pallas as pl


def kernel(x, edge_index, W1, b1, W2, b2, pW, pb, c1W, c1b, c2W, c2b, linW, linb):
    raise NotImplementedError("write your pallas kernel here")


# speedup vs baseline: 12.1985x; 12.1985x over previous
import functools

import jax
import jax.numpy as jnp
from jax import lax
from jax.experimental import pallas as pl
from jax.experimental.pallas import tpu as pltpu
from jax.experimental.pallas import tpu_sc as plsc

N = 10000
E = 320000
D = 128
H = 128
C = 64
K = 5

NC = 2
NS = 16
NW = NC * NS
EW = E // NW
ECH = 80
ENC = EW // ECH

RW = N // NW
RTAIL = N - RW * NW
RCH = 104

BN = 1000
GRID = N // BN

@functools.cache
def _mesh():
  return plsc.VectorSubcoreMesh(
      core_axis_name="c", subcore_axis_name="s", num_cores=NC, num_subcores=NS)


def _wid():
  return lax.axis_index("c") * NS + lax.axis_index("s")


NP = 640 * NS
STRIPE = NP // NS


def _deg_body(dst_hbm, out_hbm, idx_v, ones_v, tmp_v, acc_sh):
  cid = lax.axis_index("c")
  sid = lax.axis_index("s")
  base = _wid() * EW
  for i in range(ECH // 16):
    ones_v[pl.ds(i * 16, 16)] = jnp.ones((16,), jnp.float32)
    tmp_v[pl.ds(i * 16, 16)] = jnp.zeros((16,), jnp.float32)
  sb = pl.multiple_of(sid * STRIPE, 8)
  for j in range(STRIPE // ECH):
    pltpu.sync_copy(tmp_v, acc_sh.at[pl.ds(sb + j * ECH, ECH)])
  plsc.subcore_barrier()

  @pl.loop(0, ENC)
  def _(c):
    off = pl.multiple_of(base + c * ECH, ECH)
    pltpu.sync_copy(dst_hbm.at[pl.ds(off, ECH)], idx_v)
    pltpu.sync_copy(ones_v, acc_sh.at[idx_v], add=True)

  plsc.subcore_barrier()
  for j in range(STRIPE // ECH):
    pltpu.sync_copy(acc_sh.at[pl.ds(sb + j * ECH, ECH)], tmp_v)
    pltpu.sync_copy(tmp_v, out_hbm.at[pl.ds(cid * NP + sb + j * ECH, ECH)])


def _deg(dst):
  return pl.kernel(
      _deg_body,
      out_type=jax.ShapeDtypeStruct((NC * NP,), jnp.float32),
      mesh=_mesh(),
      scratch_types=[
          pltpu.VMEM((ECH,), jnp.int32),
          pltpu.VMEM((ECH,), jnp.float32),
          pltpu.VMEM((ECH,), jnp.float32),
          pltpu.VMEM_SHARED((NP,), jnp.float32),
      ],
  )(dst)


def _make_edge_body(f):
  def _edge_body(xs_hbm, src_hbm, dst_hbm, out_hbm,
                 idx_s, idx_d, rows_v, zb_v, acc_sh, sem):
    cid = lax.axis_index("c")
    sid = lax.axis_index("s")
    base = _wid() * EW
    for r in range(16):
      for cc in range(f // 16):
        zb_v[r, pl.ds(cc * 16, 16)] = jnp.zeros((16,), jnp.float32)
    sb = pl.multiple_of(sid * STRIPE, 8)

    @pl.loop(0, STRIPE // 16)
    def _(j):
      pltpu.sync_copy(zb_v, acc_sh.at[pl.ds(sb + j * 16, 16)])

    plsc.subcore_barrier()

    @pl.loop(0, ENC)
    def _(c):
      off = pl.multiple_of(base + c * ECH, ECH)
      pltpu.sync_copy(src_hbm.at[pl.ds(off, ECH)], idx_s)
      pltpu.sync_copy(dst_hbm.at[pl.ds(off, ECH)], idx_d)
      pltpu.async_copy(xs_hbm.at[idx_s], rows_v, sem).wait()
      pltpu.sync_copy(rows_v, acc_sh.at[idx_d], add=True)

    plsc.subcore_barrier()
    for j in range(STRIPE // ECH):
      pltpu.sync_copy(acc_sh.at[pl.ds(sb + j * ECH, ECH)], rows_v)
      pltpu.sync_copy(rows_v, out_hbm.at[cid, pl.ds(sb + j * ECH, ECH)])

  return _edge_body


def _edge_pass(xs, src, dst, f):
  return pl.kernel(
      _make_edge_body(f),
      out_type=jax.ShapeDtypeStruct((NC, NP, f), jnp.float32),
      mesh=_mesh(),
      compiler_params=pltpu.CompilerParams(use_tc_tiling_on_sc=False),
      scratch_types=[
          pltpu.VMEM((ECH,), jnp.int32),
          pltpu.VMEM((ECH,), jnp.int32),
          pltpu.VMEM((ECH, f), jnp.float32),
          pltpu.VMEM((16, f), jnp.float32),
          pltpu.VMEM_SHARED((NP, f), jnp.float32),
          pltpu.SemaphoreType.DMA,
      ],
  )(xs, src, dst)


def _gather_body(tab_hbm, ord_hbm, out_hbm, idx_v, rows_v, idx_t, rows_t, sem):
  w = _wid()
  base = w * RW
  for c in range(RW // RCH):
    off = pl.multiple_of(base + c * RCH, 8)
    pltpu.sync_copy(ord_hbm.at[pl.ds(off, RCH)], idx_v)
    pltpu.async_copy(tab_hbm.at[idx_v], rows_v, sem).wait()
    pltpu.sync_copy(rows_v, out_hbm.at[pl.ds(off, RCH)])

  @pl.when(w == NW - 1)
  def _():
    pltpu.sync_copy(ord_hbm.at[pl.ds(RW * NW, RTAIL)], idx_t)
    pltpu.async_copy(tab_hbm.at[idx_t], rows_t, sem).wait()
    pltpu.sync_copy(rows_t, out_hbm.at[pl.ds(RW * NW, RTAIL)])


def _gather_rows(tab, order):
  return pl.kernel(
      _gather_body,
      out_type=jax.ShapeDtypeStruct((N, C), jnp.float32),
      mesh=_mesh(),
      compiler_params=pltpu.CompilerParams(use_tc_tiling_on_sc=False),
      scratch_types=[
          pltpu.VMEM((RCH,), jnp.int32),
          pltpu.VMEM((RCH, C), jnp.float32),
          pltpu.VMEM((RTAIL,), jnp.int32),
          pltpu.VMEM((RTAIL, C), jnp.float32),
          pltpu.SemaphoreType.DMA,
      ],
  )(tab, order)


def _scatter_body(rows_hbm, ord_hbm, out_hbm, idx_v, rows_v, idx_t, rows_t, sem):
  w = _wid()
  base = w * RW
  for c in range(RW // RCH):
    off = pl.multiple_of(base + c * RCH, 8)
    pltpu.sync_copy(ord_hbm.at[pl.ds(off, RCH)], idx_v)
    pltpu.sync_copy(rows_hbm.at[pl.ds(off, RCH)], rows_v)
    pltpu.async_copy(rows_v, out_hbm.at[idx_v], sem).wait()

  @pl.when(w == NW - 1)
  def _():
    pltpu.sync_copy(ord_hbm.at[pl.ds(RW * NW, RTAIL)], idx_t)
    pltpu.sync_copy(rows_hbm.at[pl.ds(RW * NW, RTAIL)], rows_t)
    pltpu.async_copy(rows_t, out_hbm.at[idx_t], sem).wait()


def _scatter_rows(rows, order):
  return pl.kernel(
      _scatter_body,
      out_type=jax.ShapeDtypeStruct((N, C), jnp.float32),
      mesh=_mesh(),
      compiler_params=pltpu.CompilerParams(use_tc_tiling_on_sc=False),
      scratch_types=[
          pltpu.VMEM((RCH,), jnp.int32),
          pltpu.VMEM((RCH, C), jnp.float32),
          pltpu.VMEM((RTAIL,), jnp.int32),
          pltpu.VMEM((RTAIL, C), jnp.float32),
          pltpu.SemaphoreType.DMA,
      ],
  )(rows, order)


def _dinv_of(degp_ref):
  deg = degp_ref[0] + degp_ref[1] + 1.0
  return lax.rsqrt(deg)


def _xw_scale_body(x_ref, w_ref, degp_ref, o_ref):
  dinv = _dinv_of(degp_ref)
  xw = jnp.dot(x_ref[...], w_ref[...], preferred_element_type=jnp.float32)
  o_ref[...] = xw * dinv


def _layer1_body(p_ref, xs_ref, degp_ref, b1_ref, w2_ref, o_ref):
  dinv = _dinv_of(degp_ref)
  s = p_ref[0] + p_ref[1] + xs_ref[...]
  h = jnp.maximum(s * dinv + b1_ref[...], 0.0)
  o_ref[...] = jnp.dot(h, w2_ref[...],
                       preferred_element_type=jnp.float32) * dinv


def _layer2_body(p_ref, hs_ref, degp_ref, b2_ref, pw_ref, pb_ref,
                 h1_ref, g_ref):
  dinv = _dinv_of(degp_ref)
  s = p_ref[0] + p_ref[1] + hs_ref[...]
  h1 = s * dinv + b2_ref[...]
  h1_ref[...] = h1
  g_ref[...] = jnp.sum(h1 * pw_ref[...], axis=1, keepdims=True) + pb_ref[0]


def _shifted(xv, sh, io):
  r = pltpu.roll(xv, (-sh) % xv.shape[0], axis=0)
  if sh < 0:
    r = jnp.where(io < -sh, 0.0, r)
  elif sh > 0:
    r = jnp.where(io >= xv.shape[0] - sh, 0.0, r)
  return r


def _conv_body(sx_ref, gs_ref, w1_ref, b1_ref, w2_ref, b2_ref, o_ref):
  xv = sx_ref[...] * gs_ref[...]
  io = lax.broadcasted_iota(jnp.int32, (N, 1), 0)
  y = jnp.zeros((N, C), jnp.float32)
  for k in range(K):
    y = y + jnp.dot(_shifted(xv, k - 2, io), w1_ref[k],
                    preferred_element_type=jnp.float32)
  y = jnp.maximum(y + b1_ref[...], 0.0)
  z = jnp.zeros((N, C), jnp.float32)
  for k in range(K):
    z = z + jnp.dot(_shifted(y, k - 2, io), w2_ref[k],
                    preferred_element_type=jnp.float32)
  o_ref[...] = z + b2_ref[...]


def _final_body(h1_ref, h2_ref, w_ref, b_ref, o_ref):
  o_ref[...] = (
      jnp.dot(h1_ref[...], w_ref[0:C, :], preferred_element_type=jnp.float32)
      + jnp.dot(h2_ref[...], w_ref[C:2 * C, :],
                preferred_element_type=jnp.float32)
      + b_ref[...])


def _row_spec(f):
  return pl.BlockSpec((BN, f), lambda i: (i, 0))


def _full_spec(shape):
  nd = len(shape)
  return pl.BlockSpec(shape, lambda i: (0,) * nd)


_degp_spec = pl.BlockSpec((NC, BN, 1), lambda i: (0, i, 0))


def kernel(x, edge_index, W1, b1, W2, b2, pW, pb, c1W, c1b, c2W, c2b,
           linW, linb):
  src = edge_index[0]
  dst = edge_index[1]
  degp = _deg(dst).reshape(NC, NP)[:, :N].reshape(NC, N, 1)

  xs = pl.pallas_call(
      _xw_scale_body,
      grid=(GRID,),
      in_specs=[_row_spec(D), _full_spec((D, H)), _degp_spec],
      out_specs=_row_spec(H),
      out_shape=jax.ShapeDtypeStruct((N, H), jnp.float32),
  )(x, W1, degp)

  p1 = _edge_pass(xs, src, dst, H)[:, :N]

  hs = pl.pallas_call(
      _layer1_body,
      grid=(GRID,),
      in_specs=[
          pl.BlockSpec((NC, BN, H), lambda i: (0, i, 0)),
          _row_spec(H),
          _degp_spec,
          _full_spec((1, H)),
          _full_spec((H, C)),
      ],
      out_specs=_row_spec(C),
      out_shape=jax.ShapeDtypeStruct((N, C), jnp.float32),
  )(p1, xs, degp, b1.reshape(1, H), W2)

  p2 = _edge_pass(hs, src, dst, C)[:, :N]

  h1, g = pl.pallas_call(
      _layer2_body,
      grid=(GRID,),
      in_specs=[
          pl.BlockSpec((NC, BN, C), lambda i: (0, i, 0)),
          _row_spec(C),
          _degp_spec,
          _full_spec((1, C)),
          _full_spec((1, C)),
          pl.BlockSpec(memory_space=pltpu.SMEM),
      ],
      out_specs=[_row_spec(C), pl.BlockSpec((BN, 1), lambda i: (i, 0))],
      out_shape=[
          jax.ShapeDtypeStruct((N, C), jnp.float32),
          jax.ShapeDtypeStruct((N, 1), jnp.float32),
      ],
  )(p2, hs, degp, b2.reshape(1, C), pW.reshape(1, C), pb)

  g_sorted, order = lax.sort_key_val(g[:, 0], jnp.arange(N, dtype=jnp.int32),
                                     is_stable=True)

  sx = _gather_rows(h1, order)

  y2 = pl.pallas_call(
      _conv_body,
      grid=(1,),
      in_specs=[
          _full_spec((N, C)),
          _full_spec((N, 1)),
          _full_spec((K, C, C)),
          _full_spec((1, C)),
          _full_spec((K, C, C)),
          _full_spec((1, C)),
      ],
      out_specs=_full_spec((N, C)),
      out_shape=jax.ShapeDtypeStruct((N, C), jnp.float32),
  )(sx, g_sorted.reshape(N, 1), jnp.transpose(c1W, (2, 1, 0)),
    c1b.reshape(1, C), jnp.transpose(c2W, (2, 1, 0)), c2b.reshape(1, C))

  h2 = _scatter_rows(y2, order)

  out = pl.pallas_call(
      _final_body,
      grid=(GRID,),
      in_specs=[
          _row_spec(C),
          _row_spec(C),
          _full_spec((2 * C, C)),
          _full_spec((1, C)),
      ],
      out_specs=_row_spec(C),
      out_shape=jax.ShapeDtypeStruct((N, C), jnp.float32),
  )(h1, h2, linW, linb.reshape(1, C))
  return out


# speedup vs baseline: 24.3609x; 1.9970x over previous
import functools

import jax
import jax.numpy as jnp
from jax import lax
from jax.experimental import pallas as pl
from jax.experimental.pallas import tpu as pltpu
from jax.experimental.pallas import tpu_sc as plsc

N = 10000
E = 320000
D = 128
H = 128
C = 64
K = 5

NC = 2
NS = 16
NW = NC * NS
EW = E // NW
ECH = 80
ENC = EW // ECH

RW = N // NW
RTAIL = N - RW * NW
RCH = 104

BN = 1000
GRID = N // BN

@functools.cache
def _mesh():
  return plsc.VectorSubcoreMesh(
      core_axis_name="c", subcore_axis_name="s", num_cores=NC, num_subcores=NS)


def _wid():
  return lax.axis_index("c") * NS + lax.axis_index("s")


NP = 640 * NS
STRIPE = NP // NS


def _deg_body(dstr_hbm, out_hbm, idx_all, ones_v, tmp_v, acc_sh):
  cid = lax.axis_index("c")
  sid = lax.axis_index("s")
  w = _wid()
  for i in range(ECH // 16):
    ones_v[pl.ds(i * 16, 16)] = jnp.ones((16,), jnp.float32)
    tmp_v[pl.ds(i * 16, 16)] = jnp.zeros((16,), jnp.float32)
  sb = pl.multiple_of(sid * STRIPE, 8)
  for j in range(STRIPE // ECH):
    pltpu.sync_copy(tmp_v, acc_sh.at[pl.ds(sb + j * ECH, ECH)])
  pltpu.sync_copy(dstr_hbm.at[w], idx_all)
  plsc.subcore_barrier()

  @pl.loop(0, ENC)
  def _(c):
    pltpu.sync_copy(ones_v, acc_sh.at[idx_all.at[c]], add=True)

  plsc.subcore_barrier()
  for j in range(STRIPE // ECH):
    pltpu.sync_copy(acc_sh.at[pl.ds(sb + j * ECH, ECH)], tmp_v)
    pltpu.sync_copy(tmp_v, out_hbm.at[pl.ds(cid * NP + sb + j * ECH, ECH)])


def _deg(dstr):
  return pl.kernel(
      _deg_body,
      out_type=jax.ShapeDtypeStruct((NC * NP,), jnp.float32),
      mesh=_mesh(),
      scratch_types=[
          pltpu.VMEM((ENC, ECH), jnp.int32),
          pltpu.VMEM((ECH,), jnp.float32),
          pltpu.VMEM((ECH,), jnp.float32),
          pltpu.VMEM_SHARED((NP,), jnp.float32),
      ],
  )(dstr)


def _make_edge_body(f):
  def _edge_body(xs_hbm, srcr_hbm, dstr_hbm, out_hbm,
                 idx_s, idx_d, rows0, rows1, zb_v, acc_sh, sem0, sem1):
    cid = lax.axis_index("c")
    sid = lax.axis_index("s")
    w = _wid()
    for r in range(16):
      for cc in range(f // 16):
        zb_v[r, pl.ds(cc * 16, 16)] = jnp.zeros((16,), jnp.float32)
    sb = pl.multiple_of(sid * STRIPE, 8)

    @pl.loop(0, STRIPE // 16)
    def _(j):
      pltpu.sync_copy(zb_v, acc_sh.at[pl.ds(sb + j * 16, 16)])

    pltpu.sync_copy(srcr_hbm.at[w], idx_s)
    pltpu.sync_copy(dstr_hbm.at[w], idx_d)
    plsc.subcore_barrier()

    pltpu.async_copy(xs_hbm.at[idx_s.at[0]], rows0, sem0)
    pltpu.async_copy(xs_hbm.at[idx_s.at[1]], rows1, sem1)

    @pl.loop(0, (ENC - 1) // 2)
    def _(g):
      i0 = g * 2
      pltpu.make_async_copy(xs_hbm.at[idx_s.at[i0]], rows0, sem0).wait()
      pltpu.sync_copy(rows0, acc_sh.at[idx_d.at[i0]], add=True)
      pltpu.async_copy(xs_hbm.at[idx_s.at[i0 + 2]], rows0, sem0)
      i1 = i0 + 1
      pltpu.make_async_copy(xs_hbm.at[idx_s.at[i1]], rows1, sem1).wait()
      pltpu.sync_copy(rows1, acc_sh.at[idx_d.at[i1]], add=True)

      @pl.when(i1 + 2 < ENC)
      def _():
        pltpu.async_copy(xs_hbm.at[idx_s.at[i1 + 2]], rows1, sem1)

    pltpu.make_async_copy(xs_hbm.at[idx_s.at[ENC - 1]], rows0, sem0).wait()
    pltpu.sync_copy(rows0, acc_sh.at[idx_d.at[ENC - 1]], add=True)

    plsc.subcore_barrier()
    for j in range(STRIPE // ECH):
      pltpu.sync_copy(acc_sh.at[pl.ds(sb + j * ECH, ECH)], rows0)
      pltpu.sync_copy(rows0, out_hbm.at[cid, pl.ds(sb + j * ECH, ECH)])

  return _edge_body


def _edge_pass(xs, srcr, dstr, f):
  return pl.kernel(
      _make_edge_body(f),
      out_type=jax.ShapeDtypeStruct((NC, NP, f), jnp.float32),
      mesh=_mesh(),
      compiler_params=pltpu.CompilerParams(use_tc_tiling_on_sc=False),
      scratch_types=[
          pltpu.VMEM((ENC, ECH), jnp.int32),
          pltpu.VMEM((ENC, ECH), jnp.int32),
          pltpu.VMEM((ECH, f), jnp.float32),
          pltpu.VMEM((ECH, f), jnp.float32),
          pltpu.VMEM((16, f), jnp.float32),
          pltpu.VMEM_SHARED((NP, f), jnp.float32),
          pltpu.SemaphoreType.DMA,
          pltpu.SemaphoreType.DMA,
      ],
  )(xs, srcr, dstr)


def _gather_body(tab_hbm, ord_hbm, out_hbm, idx_v, rows_v, idx_t, rows_t, sem):
  w = _wid()
  base = w * RW
  for c in range(RW // RCH):
    off = pl.multiple_of(base + c * RCH, 8)
    pltpu.sync_copy(ord_hbm.at[pl.ds(off, RCH)], idx_v)
    pltpu.async_copy(tab_hbm.at[idx_v], rows_v, sem).wait()
    pltpu.sync_copy(rows_v, out_hbm.at[pl.ds(off, RCH)])

  @pl.when(w == NW - 1)
  def _():
    pltpu.sync_copy(ord_hbm.at[pl.ds(RW * NW, RTAIL)], idx_t)
    pltpu.async_copy(tab_hbm.at[idx_t], rows_t, sem).wait()
    pltpu.sync_copy(rows_t, out_hbm.at[pl.ds(RW * NW, RTAIL)])


def _gather_rows(tab, order):
  return pl.kernel(
      _gather_body,
      out_type=jax.ShapeDtypeStruct((N, C), jnp.float32),
      mesh=_mesh(),
      compiler_params=pltpu.CompilerParams(use_tc_tiling_on_sc=False),
      scratch_types=[
          pltpu.VMEM((RCH,), jnp.int32),
          pltpu.VMEM((RCH, C), jnp.float32),
          pltpu.VMEM((RTAIL,), jnp.int32),
          pltpu.VMEM((RTAIL, C), jnp.float32),
          pltpu.SemaphoreType.DMA,
      ],
  )(tab, order)


def _scatter_body(rows_hbm, ord_hbm, out_hbm, idx_v, rows_v, idx_t, rows_t, sem):
  w = _wid()
  base = w * RW
  for c in range(RW // RCH):
    off = pl.multiple_of(base + c * RCH, 8)
    pltpu.sync_copy(ord_hbm.at[pl.ds(off, RCH)], idx_v)
    pltpu.sync_copy(rows_hbm.at[pl.ds(off, RCH)], rows_v)
    pltpu.async_copy(rows_v, out_hbm.at[idx_v], sem).wait()

  @pl.when(w == NW - 1)
  def _():
    pltpu.sync_copy(ord_hbm.at[pl.ds(RW * NW, RTAIL)], idx_t)
    pltpu.sync_copy(rows_hbm.at[pl.ds(RW * NW, RTAIL)], rows_t)
    pltpu.async_copy(rows_t, out_hbm.at[idx_t], sem).wait()


def _scatter_rows(rows, order):
  return pl.kernel(
      _scatter_body,
      out_type=jax.ShapeDtypeStruct((N, C), jnp.float32),
      mesh=_mesh(),
      compiler_params=pltpu.CompilerParams(use_tc_tiling_on_sc=False),
      scratch_types=[
          pltpu.VMEM((RCH,), jnp.int32),
          pltpu.VMEM((RCH, C), jnp.float32),
          pltpu.VMEM((RTAIL,), jnp.int32),
          pltpu.VMEM((RTAIL, C), jnp.float32),
          pltpu.SemaphoreType.DMA,
      ],
  )(rows, order)


def _dinv_of(degp_ref):
  deg = degp_ref[0] + degp_ref[1] + 1.0
  return lax.rsqrt(deg)


def _xw_scale_body(x_ref, w_ref, degp_ref, o_ref):
  dinv = _dinv_of(degp_ref)
  xw = jnp.dot(x_ref[...], w_ref[...], preferred_element_type=jnp.float32)
  o_ref[...] = xw * dinv


def _layer1_body(p_ref, xs_ref, degp_ref, b1_ref, w2_ref, o_ref):
  dinv = _dinv_of(degp_ref)
  s = p_ref[0] + p_ref[1] + xs_ref[...]
  h = jnp.maximum(s * dinv + b1_ref[...], 0.0)
  o_ref[...] = jnp.dot(h, w2_ref[...],
                       preferred_element_type=jnp.float32) * dinv


def _layer2_body(p_ref, hs_ref, degp_ref, b2_ref, pw_ref, pb_ref,
                 h1_ref, g_ref):
  dinv = _dinv_of(degp_ref)
  s = p_ref[0] + p_ref[1] + hs_ref[...]
  h1 = s * dinv + b2_ref[...]
  h1_ref[...] = h1
  g_ref[...] = jnp.sum(h1 * pw_ref[...], axis=1, keepdims=True) + pb_ref[0]


def _shifted(xv, sh, io):
  r = pltpu.roll(xv, (-sh) % xv.shape[0], axis=0)
  if sh < 0:
    r = jnp.where(io < -sh, 0.0, r)
  elif sh > 0:
    r = jnp.where(io >= xv.shape[0] - sh, 0.0, r)
  return r


def _conv_body(sx_ref, gs_ref, w1_ref, b1_ref, w2_ref, b2_ref, o_ref):
  xv = sx_ref[...] * gs_ref[...]
  io = lax.broadcasted_iota(jnp.int32, (N, 1), 0)
  y = jnp.zeros((N, C), jnp.float32)
  for k in range(K):
    y = y + jnp.dot(_shifted(xv, k - 2, io), w1_ref[k],
                    preferred_element_type=jnp.float32)
  y = jnp.maximum(y + b1_ref[...], 0.0)
  z = jnp.zeros((N, C), jnp.float32)
  for k in range(K):
    z = z + jnp.dot(_shifted(y, k - 2, io), w2_ref[k],
                    preferred_element_type=jnp.float32)
  o_ref[...] = z + b2_ref[...]


def _final_body(h1_ref, h2_ref, w_ref, b_ref, o_ref):
  o_ref[...] = (
      jnp.dot(h1_ref[...], w_ref[0:C, :], preferred_element_type=jnp.float32)
      + jnp.dot(h2_ref[...], w_ref[C:2 * C, :],
                preferred_element_type=jnp.float32)
      + b_ref[...])


def _row_spec(f):
  return pl.BlockSpec((BN, f), lambda i: (i, 0))


def _full_spec(shape):
  nd = len(shape)
  return pl.BlockSpec(shape, lambda i: (0,) * nd)


_degp_spec = pl.BlockSpec((NC, BN, 1), lambda i: (0, i, 0))


def kernel(x, edge_index, W1, b1, W2, b2, pW, pb, c1W, c1b, c2W, c2b,
           linW, linb):
  srcr = edge_index[0].reshape(NW, ENC, ECH)
  dstr = edge_index[1].reshape(NW, ENC, ECH)
  degp = _deg(dstr).reshape(NC, NP)[:, :N].reshape(NC, N, 1)

  xs = pl.pallas_call(
      _xw_scale_body,
      grid=(GRID,),
      in_specs=[_row_spec(D), _full_spec((D, H)), _degp_spec],
      out_specs=_row_spec(H),
      out_shape=jax.ShapeDtypeStruct((N, H), jnp.float32),
  )(x, W1, degp)

  p1 = _edge_pass(xs, srcr, dstr, H)[:, :N]

  hs = pl.pallas_call(
      _layer1_body,
      grid=(GRID,),
      in_specs=[
          pl.BlockSpec((NC, BN, H), lambda i: (0, i, 0)),
          _row_spec(H),
          _degp_spec,
          _full_spec((1, H)),
          _full_spec((H, C)),
      ],
      out_specs=_row_spec(C),
      out_shape=jax.ShapeDtypeStruct((N, C), jnp.float32),
  )(p1, xs, degp, b1.reshape(1, H), W2)

  p2 = _edge_pass(hs, srcr, dstr, C)[:, :N]

  h1, g = pl.pallas_call(
      _layer2_body,
      grid=(GRID,),
      in_specs=[
          pl.BlockSpec((NC, BN, C), lambda i: (0, i, 0)),
          _row_spec(C),
          _degp_spec,
          _full_spec((1, C)),
          _full_spec((1, C)),
          pl.BlockSpec(memory_space=pltpu.SMEM),
      ],
      out_specs=[_row_spec(C), pl.BlockSpec((BN, 1), lambda i: (i, 0))],
      out_shape=[
          jax.ShapeDtypeStruct((N, C), jnp.float32),
          jax.ShapeDtypeStruct((N, 1), jnp.float32),
      ],
  )(p2, hs, degp, b2.reshape(1, C), pW.reshape(1, C), pb)

  g_sorted, order = lax.sort_key_val(g[:, 0], jnp.arange(N, dtype=jnp.int32),
                                     is_stable=True)

  sx = _gather_rows(h1, order)

  y2 = pl.pallas_call(
      _conv_body,
      grid=(1,),
      in_specs=[
          _full_spec((N, C)),
          _full_spec((N, 1)),
          _full_spec((K, C, C)),
          _full_spec((1, C)),
          _full_spec((K, C, C)),
          _full_spec((1, C)),
      ],
      out_specs=_full_spec((N, C)),
      out_shape=jax.ShapeDtypeStruct((N, C), jnp.float32),
  )(sx, g_sorted.reshape(N, 1), jnp.transpose(c1W, (2, 1, 0)),
    c1b.reshape(1, C), jnp.transpose(c2W, (2, 1, 0)), c2b.reshape(1, C))

  h2 = _scatter_rows(y2, order)

  out = pl.pallas_call(
      _final_body,
      grid=(GRID,),
      in_specs=[
          _row_spec(C),
          _row_spec(C),
          _full_spec((2 * C, C)),
          _full_spec((1, C)),
      ],
      out_specs=_row_spec(C),
      out_shape=jax.ShapeDtypeStruct((N, C), jnp.float32),
  )(h1, h2, linW, linb.reshape(1, C))
  return out


# speedup vs baseline: 25.5761x; 1.0499x over previous
import functools

import jax
import jax.numpy as jnp
from jax import lax
from jax.experimental import pallas as pl
from jax.experimental.pallas import tpu as pltpu
from jax.experimental.pallas import tpu_sc as plsc

N = 10000
E = 320000
D = 128
H = 128
C = 64
K = 5

NC = 2
NS = 16
NW = NC * NS
EW = E // NW
ECH = 80
ENC = EW // ECH

RW = N // NW
RTAIL = N - RW * NW
RCH = 104

BN = 1000
GRID = N // BN

@functools.cache
def _mesh():
  return plsc.VectorSubcoreMesh(
      core_axis_name="c", subcore_axis_name="s", num_cores=NC, num_subcores=NS)


def _wid():
  return lax.axis_index("c") * NS + lax.axis_index("s")


NP = 640 * NS
STRIPE = NP // NS


def _deg_body(dstr_hbm, out_hbm, idx_all, ones_v, tmp_v, acc_sh):
  cid = lax.axis_index("c")
  sid = lax.axis_index("s")
  w = _wid()
  for i in range(ECH // 16):
    ones_v[pl.ds(i * 16, 16)] = jnp.ones((16,), jnp.float32)
    tmp_v[pl.ds(i * 16, 16)] = jnp.zeros((16,), jnp.float32)
  sb = pl.multiple_of(sid * STRIPE, 8)
  for j in range(STRIPE // ECH):
    pltpu.sync_copy(tmp_v, acc_sh.at[pl.ds(sb + j * ECH, ECH)])
  pltpu.sync_copy(dstr_hbm.at[w], idx_all)
  plsc.subcore_barrier()

  @pl.loop(0, ENC)
  def _(c):
    pltpu.sync_copy(ones_v, acc_sh.at[idx_all.at[c]], add=True)

  plsc.subcore_barrier()
  for j in range(STRIPE // ECH):
    pltpu.sync_copy(acc_sh.at[pl.ds(sb + j * ECH, ECH)], tmp_v)
    pltpu.sync_copy(tmp_v, out_hbm.at[pl.ds(cid * NP + sb + j * ECH, ECH)])


def _deg(dstr):
  return pl.kernel(
      _deg_body,
      out_type=jax.ShapeDtypeStruct((NC * NP,), jnp.float32),
      mesh=_mesh(),
      scratch_types=[
          pltpu.VMEM((ENC, ECH), jnp.int32),
          pltpu.VMEM((ECH,), jnp.float32),
          pltpu.VMEM((ECH,), jnp.float32),
          pltpu.VMEM_SHARED((NP,), jnp.float32),
      ],
  )(dstr)


def _make_edge_body(f):
  def _edge_body(xs_hbm, srcr_hbm, dstr_hbm, out_hbm,
                 idx_s, idx_d, rows0, rows1, zb_v, acc_sh,
                 sem0, sem1):
    cid = lax.axis_index("c")
    sid = lax.axis_index("s")
    w = _wid()
    for r in range(16):
      for cc in range(f // 16):
        zb_v[r, pl.ds(cc * 16, 16)] = jnp.zeros((16,), jnp.float32)
    sb = pl.multiple_of(sid * STRIPE, 8)

    @pl.loop(0, STRIPE // 16)
    def _(j):
      pltpu.sync_copy(zb_v, acc_sh.at[pl.ds(sb + j * 16, 16)])

    pltpu.sync_copy(srcr_hbm.at[w], idx_s)
    pltpu.sync_copy(dstr_hbm.at[w], idx_d)
    plsc.subcore_barrier()

    rows = (rows0, rows1)
    sems = (sem0, sem1)
    for b in range(2):
      pltpu.async_copy(xs_hbm.at[idx_s.at[b]], rows[b], sems[b])

    @pl.loop(0, ENC // 2)
    def _(g):
      for b in range(2):
        i = g * 2 + b
        pltpu.make_async_copy(xs_hbm.at[idx_s.at[i]], rows[b], sems[b]).wait()
        pltpu.sync_copy(rows[b], acc_sh.at[idx_d.at[i]], add=True)
        if (ENC // 2) * 2 + b < ENC:
          pltpu.async_copy(xs_hbm.at[idx_s.at[i + 2]], rows[b], sems[b])
        else:
          @pl.when(i + 2 < ENC)
          def _():
            pltpu.async_copy(xs_hbm.at[idx_s.at[i + 2]], rows[b], sems[b])

    for b in range(ENC - (ENC // 2) * 2):
      i = (ENC // 2) * 2 + b
      pltpu.make_async_copy(xs_hbm.at[idx_s.at[i]], rows[b], sems[b]).wait()
      pltpu.sync_copy(rows[b], acc_sh.at[idx_d.at[i]], add=True)

    plsc.subcore_barrier()
    for j in range(STRIPE // ECH):
      pltpu.sync_copy(acc_sh.at[pl.ds(sb + j * ECH, ECH)], rows0)
      pltpu.sync_copy(rows0, out_hbm.at[cid, pl.ds(sb + j * ECH, ECH)])

  return _edge_body


def _edge_pass(xs, srcr, dstr, f):
  return pl.kernel(
      _make_edge_body(f),
      out_type=jax.ShapeDtypeStruct((NC, NP, f), jnp.float32),
      mesh=_mesh(),
      compiler_params=pltpu.CompilerParams(use_tc_tiling_on_sc=False),
      scratch_types=[
          pltpu.VMEM((ENC, ECH), jnp.int32),
          pltpu.VMEM((ENC, ECH), jnp.int32),
          pltpu.VMEM((ECH, f), jnp.float32),
          pltpu.VMEM((ECH, f), jnp.float32),
          pltpu.VMEM((16, f), jnp.float32),
          pltpu.VMEM_SHARED((NP, f), jnp.float32),
          pltpu.SemaphoreType.DMA,
          pltpu.SemaphoreType.DMA,
      ],
  )(xs, srcr, dstr)


def _gather_body(tab_hbm, ord_hbm, out_hbm, idx_v, rows_v, idx_t, rows_t, sem):
  w = _wid()
  base = w * RW
  for c in range(RW // RCH):
    off = pl.multiple_of(base + c * RCH, 8)
    pltpu.sync_copy(ord_hbm.at[pl.ds(off, RCH)], idx_v)
    pltpu.async_copy(tab_hbm.at[idx_v], rows_v, sem).wait()
    pltpu.sync_copy(rows_v, out_hbm.at[pl.ds(off, RCH)])

  @pl.when(w == NW - 1)
  def _():
    pltpu.sync_copy(ord_hbm.at[pl.ds(RW * NW, RTAIL)], idx_t)
    pltpu.async_copy(tab_hbm.at[idx_t], rows_t, sem).wait()
    pltpu.sync_copy(rows_t, out_hbm.at[pl.ds(RW * NW, RTAIL)])


def _gather_rows(tab, order):
  return pl.kernel(
      _gather_body,
      out_type=jax.ShapeDtypeStruct((N, C), jnp.float32),
      mesh=_mesh(),
      compiler_params=pltpu.CompilerParams(use_tc_tiling_on_sc=False),
      scratch_types=[
          pltpu.VMEM((RCH,), jnp.int32),
          pltpu.VMEM((RCH, C), jnp.float32),
          pltpu.VMEM((RTAIL,), jnp.int32),
          pltpu.VMEM((RTAIL, C), jnp.float32),
          pltpu.SemaphoreType.DMA,
      ],
  )(tab, order)


def _scatter_body(rows_hbm, ord_hbm, out_hbm, idx_v, rows_v, idx_t, rows_t, sem):
  w = _wid()
  base = w * RW
  for c in range(RW // RCH):
    off = pl.multiple_of(base + c * RCH, 8)
    pltpu.sync_copy(ord_hbm.at[pl.ds(off, RCH)], idx_v)
    pltpu.sync_copy(rows_hbm.at[pl.ds(off, RCH)], rows_v)
    pltpu.async_copy(rows_v, out_hbm.at[idx_v], sem).wait()

  @pl.when(w == NW - 1)
  def _():
    pltpu.sync_copy(ord_hbm.at[pl.ds(RW * NW, RTAIL)], idx_t)
    pltpu.sync_copy(rows_hbm.at[pl.ds(RW * NW, RTAIL)], rows_t)
    pltpu.async_copy(rows_t, out_hbm.at[idx_t], sem).wait()


def _scatter_rows(rows, order):
  return pl.kernel(
      _scatter_body,
      out_type=jax.ShapeDtypeStruct((N, C), jnp.float32),
      mesh=_mesh(),
      compiler_params=pltpu.CompilerParams(use_tc_tiling_on_sc=False),
      scratch_types=[
          pltpu.VMEM((RCH,), jnp.int32),
          pltpu.VMEM((RCH, C), jnp.float32),
          pltpu.VMEM((RTAIL,), jnp.int32),
          pltpu.VMEM((RTAIL, C), jnp.float32),
          pltpu.SemaphoreType.DMA,
      ],
  )(rows, order)


def _dinv_of(degp_ref):
  deg = degp_ref[0] + degp_ref[1] + 1.0
  return lax.rsqrt(deg)


def _xw_scale_body(x_ref, w_ref, degp_ref, o_ref):
  dinv = _dinv_of(degp_ref)
  xw = jnp.dot(x_ref[...], w_ref[...], preferred_element_type=jnp.float32)
  o_ref[...] = xw * dinv


def _layer1_body(p_ref, xs_ref, degp_ref, b1_ref, w2_ref, o_ref):
  dinv = _dinv_of(degp_ref)
  s = p_ref[0] + p_ref[1] + xs_ref[...]
  h = jnp.maximum(s * dinv + b1_ref[...], 0.0)
  o_ref[...] = jnp.dot(h, w2_ref[...],
                       preferred_element_type=jnp.float32) * dinv


def _layer2_body(p_ref, hs_ref, degp_ref, b2_ref, h1_ref):
  dinv = _dinv_of(degp_ref)
  s = p_ref[0] + p_ref[1] + hs_ref[...]
  h1_ref[...] = s * dinv + b2_ref[...]


def _shifted(xv, sh, io):
  r = pltpu.roll(xv, (-sh) % xv.shape[0], axis=0)
  if sh < 0:
    r = jnp.where(io < -sh, 0.0, r)
  elif sh > 0:
    r = jnp.where(io >= xv.shape[0] - sh, 0.0, r)
  return r


def _conv_body(sx_ref, gs_ref, w1_ref, b1_ref, w2_ref, b2_ref,
               lw_ref, lb_ref, o_ref):
  sx = sx_ref[...]
  xv = sx * gs_ref[...]
  io = lax.broadcasted_iota(jnp.int32, (N, 1), 0)
  y = jnp.zeros((N, C), jnp.float32)
  for k in range(K):
    y = y + jnp.dot(_shifted(xv, k - 2, io), w1_ref[k],
                    preferred_element_type=jnp.float32)
  y = jnp.maximum(y + b1_ref[...], 0.0)
  z = jnp.zeros((N, C), jnp.float32)
  for k in range(K):
    z = z + jnp.dot(_shifted(y, k - 2, io), w2_ref[k],
                    preferred_element_type=jnp.float32)
  y2 = z + b2_ref[...]
  o_ref[...] = (
      jnp.dot(sx, lw_ref[0:C, :], preferred_element_type=jnp.float32)
      + jnp.dot(y2, lw_ref[C:2 * C, :], preferred_element_type=jnp.float32)
      + lb_ref[...])


def _row_spec(f):
  return pl.BlockSpec((BN, f), lambda i: (i, 0))


def _full_spec(shape):
  nd = len(shape)
  return pl.BlockSpec(shape, lambda i: (0,) * nd)


_degp_spec = pl.BlockSpec((NC, BN, 1), lambda i: (0, i, 0))


def kernel(x, edge_index, W1, b1, W2, b2, pW, pb, c1W, c1b, c2W, c2b,
           linW, linb):
  srcr = edge_index[0].reshape(NW, ENC, ECH)
  dstr = edge_index[1].reshape(NW, ENC, ECH)
  degp = _deg(dstr).reshape(NC, NP)[:, :N].reshape(NC, N, 1)

  xs = pl.pallas_call(
      _xw_scale_body,
      grid=(GRID,),
      in_specs=[_row_spec(D), _full_spec((D, H)), _degp_spec],
      out_specs=_row_spec(H),
      out_shape=jax.ShapeDtypeStruct((N, H), jnp.float32),
  )(x, W1, degp)

  p1 = _edge_pass(xs, srcr, dstr, H)[:, :N]

  hs = pl.pallas_call(
      _layer1_body,
      grid=(GRID,),
      in_specs=[
          pl.BlockSpec((NC, BN, H), lambda i: (0, i, 0)),
          _row_spec(H),
          _degp_spec,
          _full_spec((1, H)),
          _full_spec((H, C)),
      ],
      out_specs=_row_spec(C),
      out_shape=jax.ShapeDtypeStruct((N, C), jnp.float32),
  )(p1, xs, degp, b1.reshape(1, H), W2)

  p2 = _edge_pass(hs, srcr, dstr, C)[:, :N]

  h1 = pl.pallas_call(
      _layer2_body,
      grid=(GRID,),
      in_specs=[
          pl.BlockSpec((NC, BN, C), lambda i: (0, i, 0)),
          _row_spec(C),
          _degp_spec,
          _full_spec((1, C)),
      ],
      out_specs=_row_spec(C),
      out_shape=jax.ShapeDtypeStruct((N, C), jnp.float32),
  )(p2, hs, degp, b2.reshape(1, C))

  g = h1 @ pW + pb
  g_sorted, order = lax.sort_key_val(g[:, 0], jnp.arange(N, dtype=jnp.int32),
                                     is_stable=True)

  sx = _gather_rows(h1, order)

  ys = pl.pallas_call(
      _conv_body,
      grid=(1,),
      in_specs=[
          _full_spec((N, C)),
          _full_spec((N, 1)),
          _full_spec((K, C, C)),
          _full_spec((1, C)),
          _full_spec((K, C, C)),
          _full_spec((1, C)),
          _full_spec((2 * C, C)),
          _full_spec((1, C)),
      ],
      out_specs=_full_spec((N, C)),
      out_shape=jax.ShapeDtypeStruct((N, C), jnp.float32),
  )(sx, g_sorted.reshape(N, 1), jnp.transpose(c1W, (2, 1, 0)),
    c1b.reshape(1, C), jnp.transpose(c2W, (2, 1, 0)), c2b.reshape(1, C),
    linW, linb.reshape(1, C))

  return _scatter_rows(ys, order)
